# baseline (speedup 1.0000x reference)
import functools

import jax
import jax.numpy as jnp
from jax import lax
from jax.experimental import pallas as pl
from jax.experimental.pallas import tpu as pltpu
from jax.experimental.pallas import tpu_sc as plsc

L = 16
F = 2
B = 262144
NCORE = 2
NSUB = 16
NW = NCORE * NSUB
PTS = B // NW
C = 512
NCHUNK = PTS // C

SL = [int(16 * 1.26 ** l) for l in range(L)]
WREAL = [s // 2 + 2 for s in SL[:15]] + [257]
WPAD = WREAL[:15] + [258]

_offs = []
_off = 0
for _w in WPAD:
    _offs.append(_off)
    _off += -((_w * _w) // -8) * 8
TOTAL_WORDS = _off
NA = _offs[14]
NB = TOTAL_WORDS - NA

NC32 = L * F
PASS_A = list(range(14))
PASS_B = [14, 15]


def _pack_tables(grid_table):
    flats_a, flats_b = [], []
    for l in range(L):
        wr, wp = WREAL[l], WPAD[l]
        f0 = grid_table[l, 0, 255:255 + wr, 255:255 + wr]
        f1 = grid_table[l, 1, 255:255 + wr, 255:255 + wr]
        u0 = jax.lax.bitcast_convert_type(f0.astype(jnp.bfloat16),
                                          jnp.uint16).astype(jnp.uint32)
        u1 = jax.lax.bitcast_convert_type(f1.astype(jnp.bfloat16),
                                          jnp.uint16).astype(jnp.uint32)
        words = jax.lax.bitcast_convert_type(u0 | (u1 << 16), jnp.int32)
        if wp != wr:
            words = jnp.pad(words, ((0, wp - wr), (0, wp - wr)))
        words = words.reshape(-1)
        pad = -((wp * wp) // -8) * 8 - wp * wp
        if pad:
            words = jnp.pad(words, (0, pad))
        (flats_a if l < 14 else flats_b).append(words)
    return jnp.concatenate(flats_a), jnp.concatenate(flats_b)


def _f32_lo(v):
    return plsc.bitcast(v << 16, jnp.float32)


def _f32_hi(v):
    return plsc.bitcast(v & jnp.int32(-65536), jnp.float32)


def _body(xt_ref, tbla_ref, tblb_ref, out_ref, tbl_v, x_v, o_v):
    cid = lax.axis_index("c")
    sid = lax.axis_index("s")
    base = (sid * NCORE + cid) * PTS

    def run_pass(tbl_hbm, nwords, levels, off0, rmw):
        pltpu.sync_copy(tbl_hbm, tbl_v.at[pl.ds(0, nwords)])

        def chunk_body(k, _):
            rowbase = base + k * C
            pltpu.sync_copy(xt_ref.at[pl.ds(rowbase * 2, C * 2)], x_v)
            if rmw:
                pltpu.sync_copy(out_ref.at[pl.ds(rowbase * NC32, C * NC32)], o_v)

            def vec_body(i, _2):
                p = i * 16
                xi = lax.iota(jnp.int32, 16) * 2 + p * 2
                xs = plsc.load_gather(x_v, [xi])
                ys = plsc.load_gather(x_v, [xi + 1])
                rbase = lax.iota(jnp.int32, 16) * NC32 + p * NC32
                for l in levels:
                    w = WPAD[l]
                    c_l = SL[l] / 2.0
                    k_l = (_offs[l] - off0) - 255 * w - 255
                    ix = xs * c_l + 255.5
                    iy = ys * c_l + 255.5
                    x0 = ix.astype(jnp.int32)
                    y0 = iy.astype(jnp.int32)
                    fx = ix - x0.astype(jnp.float32)
                    fy = iy - y0.astype(jnp.float32)
                    gx = 1.0 - fx
                    gy = 1.0 - fy
                    i00 = y0 * w + x0 + k_l
                    v00 = plsc.load_gather(tbl_v, [i00])
                    v01 = plsc.load_gather(tbl_v, [i00 + 1])
                    v10 = plsc.load_gather(tbl_v, [i00 + w])
                    v11 = plsc.load_gather(tbl_v, [i00 + (w + 1)])
                    w00 = gx * gy
                    w01 = fx * gy
                    w10 = gx * fy
                    w11 = fx * fy
                    a0 = ((w00 * _f32_lo(v00) + w01 * _f32_lo(v01))
                          + (w10 * _f32_lo(v10) + w11 * _f32_lo(v11)))
                    a1 = ((w00 * _f32_hi(v00) + w01 * _f32_hi(v01))
                          + (w10 * _f32_hi(v10) + w11 * _f32_hi(v11)))
                    plsc.store_scatter(o_v, [rbase + l], a0)
                    plsc.store_scatter(o_v, [rbase + (L + l)], a1)
                return 0

            lax.fori_loop(0, C // 16, vec_body, 0)
            pltpu.sync_copy(o_v, out_ref.at[pl.ds(rowbase * NC32, C * NC32)])
            return 0

        lax.fori_loop(0, NCHUNK, chunk_body, 0)

    def zero_body(i, _):
        rbase = lax.iota(jnp.int32, 16) * NC32 + i * 16 * NC32
        z = jnp.zeros((16,), jnp.float32)
        for cc in (14, 15, 30, 31):
            plsc.store_scatter(o_v, [rbase + cc], z)
        return 0

    lax.fori_loop(0, C // 16, zero_body, 0)
    run_pass(tbla_ref, NA, PASS_A, 0, rmw=False)
    run_pass(tblb_ref, NB, PASS_B, _offs[14], rmw=True)


@jax.jit
def kernel(x, grid_table):
    tbl_a, tbl_b = _pack_tables(grid_table)
    xt = x.reshape(-1)
    mesh = plsc.VectorSubcoreMesh(core_axis_name="c", subcore_axis_name="s")
    fn = pl.kernel(
        _body,
        out_type=jax.ShapeDtypeStruct((B * L * F,), jnp.float32),
        mesh=mesh,
        compiler_params=pltpu.CompilerParams(needs_layout_passes=False),
        scratch_types=[
            pltpu.VMEM((NB,), jnp.int32),
            pltpu.VMEM((C * 2,), jnp.float32),
            pltpu.VMEM((C * L * F,), jnp.float32),
        ],
    )
    return fn(xt, tbl_a, tbl_b).reshape(B, L * F)


# speedup vs baseline: 1.0759x; 1.0759x over previous
import jax
import jax.numpy as jnp
from jax import lax
from jax.experimental import pallas as pl
from jax.experimental.pallas import tpu as pltpu
from jax.experimental.pallas import tpu_sc as plsc

L = 16
F = 2
NCOL = L * F
B = 262144
NCORE = 2
NSUB = 16
NW = NCORE * NSUB
PTS = B // NW
C = 256
NCHUNK = PTS // C
NJ = NCHUNK // 2

SL = [int(16 * 1.26 ** l) for l in range(L)]
WREAL = [s // 2 + 2 for s in SL[:15]] + [257]
WPAD = WREAL[:15] + [258]

_offs = []
_off = 0
for _w in WPAD:
    _offs.append(_off)
    _off += -((_w * _w) // -8) * 8
TOTAL_WORDS = _off
NA = _offs[14]
NB = TOTAL_WORDS - NA

PASS_A = list(range(14))
PASS_B = [14, 15]


def _pack_tables(grid_table):
    flats_a, flats_b = [], []
    for l in range(L):
        wr, wp = WREAL[l], WPAD[l]
        blk = grid_table[l, :, 255:255 + wr, 255:255 + wr]
        blk = jnp.transpose(blk, (1, 2, 0)).astype(jnp.bfloat16)
        if wp != wr:
            blk = jnp.pad(blk, ((0, wp - wr), (0, wp - wr), (0, 0)))
        words = jax.lax.bitcast_convert_type(blk, jnp.int32).reshape(-1)
        pad = -((wp * wp) // -8) * 8 - wp * wp
        if pad:
            words = jnp.pad(words, (0, pad))
        (flats_a if l < 14 else flats_b).append(words)
    return jnp.concatenate(flats_a), jnp.concatenate(flats_b)


def _f32_lo(v):
    return plsc.bitcast(v << 16, jnp.float32)


def _f32_hi(v):
    return plsc.bitcast(v & jnp.int32(-65536), jnp.float32)


def _body(xt_ref, tbla_ref, tblb_ref, out_ref,
          tbl_v, x0_v, x1_v, o0_v, o1_v,
          sx0, sx1, sw0, sw1, sr0, sr1):
    cid = lax.axis_index("c")
    sid = lax.axis_index("s")
    base = (sid * NCORE + cid) * PTS

    x_bufs = (x0_v, x1_v)
    o_bufs = (o0_v, o1_v)
    sx = (sx0, sx1)
    sw = (sw0, sw1)
    sr = (sr0, sr1)

    def x_src(k):
        return xt_ref.at[pl.ds((base + k * C) * 2, C * 2)]

    def out_rows(k):
        return out_ref.at[pl.ds((base + k * C) * NCOL, C * NCOL)]

    def compute(x_v, o_v, levels, off0):
        def vec_body(i, _):
            p = i * 16
            xi = lax.iota(jnp.int32, 16) * 2 + p * 2
            xs = plsc.load_gather(x_v, [xi])
            ys = plsc.load_gather(x_v, [xi + 1])
            rbase = lax.iota(jnp.int32, 16) * NCOL + p * NCOL
            for l in levels:
                w = WPAD[l]
                c_l = SL[l] / 2.0
                k_l = (_offs[l] - off0) - 255 * w - 255
                ix = xs * c_l + 255.5
                iy = ys * c_l + 255.5
                x0 = ix.astype(jnp.int32)
                y0 = iy.astype(jnp.int32)
                fx = ix - x0.astype(jnp.float32)
                fy = iy - y0.astype(jnp.float32)
                gx = 1.0 - fx
                gy = 1.0 - fy
                i00 = y0 * w + x0 + k_l
                v00 = plsc.load_gather(tbl_v, [i00])
                v01 = plsc.load_gather(tbl_v, [i00 + 1])
                v10 = plsc.load_gather(tbl_v, [i00 + w])
                v11 = plsc.load_gather(tbl_v, [i00 + (w + 1)])
                w00 = gx * gy
                w01 = fx * gy
                w10 = gx * fy
                w11 = fx * fy
                a0 = ((w00 * _f32_lo(v00) + w01 * _f32_lo(v01))
                      + (w10 * _f32_lo(v10) + w11 * _f32_lo(v11)))
                a1 = ((w00 * _f32_hi(v00) + w01 * _f32_hi(v01))
                      + (w10 * _f32_hi(v10) + w11 * _f32_hi(v11)))
                plsc.store_scatter(o_v, [rbase + l], a0)
                plsc.store_scatter(o_v, [rbase + (L + l)], a1)
            return 0

        lax.fori_loop(0, C // 16, vec_body, 0)

    def run_pass(tbl_hbm, nwords, levels, off0, rmw):
        pltpu.sync_copy(tbl_hbm, tbl_v.at[pl.ds(0, nwords)])
        pltpu.async_copy(x_src(0), x_bufs[0], sx[0])
        pltpu.async_copy(x_src(1), x_bufs[1], sx[1])
        if rmw:
            pltpu.async_copy(out_rows(0), o_bufs[0], sr[0])
            pltpu.async_copy(out_rows(1), o_bufs[1], sr[1])

        def half(k, b):
            pltpu.make_async_copy(x_src(k), x_bufs[b], sx[b]).wait()
            if rmw:
                pltpu.make_async_copy(out_rows(k), o_bufs[b], sr[b]).wait()
            else:
                @pl.when(k >= 2)
                def _():
                    pltpu.make_async_copy(o_bufs[b], out_rows(k - 2),
                                          sw[b]).wait()
            compute(x_bufs[b], o_bufs[b], levels, off0)
            pltpu.async_copy(o_bufs[b], out_rows(k), sw[b])

            @pl.when(k + 2 <= NCHUNK - 1)
            def _():
                pltpu.async_copy(x_src(k + 2), x_bufs[b], sx[b])
                if rmw:
                    pltpu.make_async_copy(o_bufs[b], out_rows(k), sw[b]).wait()
                    pltpu.async_copy(out_rows(k + 2), o_bufs[b], sr[b])

        def jbody(j, _):
            half(j * 2, 0)
            half(j * 2 + 1, 1)
            return 0

        lax.fori_loop(0, NJ, jbody, 0)
        pltpu.make_async_copy(o_bufs[0], out_rows(NCHUNK - 2), sw[0]).wait()
        pltpu.make_async_copy(o_bufs[1], out_rows(NCHUNK - 1), sw[1]).wait()

    def zero_body(i, _):
        rbase = lax.iota(jnp.int32, 16) * NCOL + i * 16 * NCOL
        z = jnp.zeros((16,), jnp.float32)
        for cc in list(range(14)) + list(range(16, 30)):
            plsc.store_scatter(o0_v, [rbase + cc], z)
            plsc.store_scatter(o1_v, [rbase + cc], z)
        return 0

    lax.fori_loop(0, C // 16, zero_body, 0)
    run_pass(tblb_ref, NB, PASS_B, _offs[14], rmw=False)
    run_pass(tbla_ref, NA, PASS_A, 0, rmw=True)


@jax.jit
def kernel(x, grid_table):
    tbl_a, tbl_b = _pack_tables(grid_table)
    xt = x.reshape(-1)
    mesh = plsc.VectorSubcoreMesh(core_axis_name="c", subcore_axis_name="s")
    fn = pl.kernel(
        _body,
        out_type=jax.ShapeDtypeStruct((B * NCOL,), jnp.float32),
        mesh=mesh,
        compiler_params=pltpu.CompilerParams(needs_layout_passes=False),
        scratch_types=[
            pltpu.VMEM((NB,), jnp.int32),
            pltpu.VMEM((C * 2,), jnp.float32),
            pltpu.VMEM((C * 2,), jnp.float32),
            pltpu.VMEM((C * NCOL,), jnp.float32),
            pltpu.VMEM((C * NCOL,), jnp.float32),
            pltpu.SemaphoreType.DMA,
            pltpu.SemaphoreType.DMA,
            pltpu.SemaphoreType.DMA,
            pltpu.SemaphoreType.DMA,
            pltpu.SemaphoreType.DMA,
            pltpu.SemaphoreType.DMA,
        ],
    )
    return fn(xt, tbl_a, tbl_b).reshape(B, NCOL)


# speedup vs baseline: 3.3087x; 3.0751x over previous
import jax
import jax.numpy as jnp
from jax import lax
from jax.experimental import pallas as pl
from jax.experimental.pallas import tpu as pltpu
from jax.experimental.pallas import tpu_sc as plsc

L = 16
F = 2
NCOL = L * F
B = 262144
NCORE = 2
NSUB = 16
NW = NCORE * NSUB
PTS = B // NW
C = 256
NCHUNK = PTS // C
NJ = NCHUNK // 2

SL = [int(16 * 1.26 ** l) for l in range(L)]
WREAL = [s // 2 + 2 for s in SL[:15]] + [257]
WPAD = WREAL[:15] + [258]

_offs = []
_off = 0
for _w in WPAD:
    _offs.append(_off)
    _off += -((_w * _w) // -8) * 8
TOTAL_WORDS = _off
NA = _offs[14]
NB = TOTAL_WORDS - NA

PASS_A = list(range(14))
PASS_B = [14, 15]


def _pack_tables(grid_table):
    flats_a, flats_b = [], []
    for l in range(L):
        wr, wp = WREAL[l], WPAD[l]
        blk = grid_table[l, :, 255:255 + wr, 255:255 + wr]
        blk = jnp.transpose(blk, (1, 2, 0)).astype(jnp.bfloat16)
        if wp != wr:
            blk = jnp.pad(blk, ((0, wp - wr), (0, wp - wr), (0, 0)))
        words = jax.lax.bitcast_convert_type(blk, jnp.int32).reshape(-1)
        pad = -((wp * wp) // -8) * 8 - wp * wp
        if pad:
            words = jnp.pad(words, (0, pad))
        (flats_a if l < 14 else flats_b).append(words)
    return jnp.concatenate(flats_a), jnp.concatenate(flats_b)


def _f32_lo(v):
    return plsc.bitcast(v << 16, jnp.float32)


def _f32_hi(v):
    return plsc.bitcast(v & jnp.int32(-65536), jnp.float32)


def _body(xt_ref, tbla_ref, tblb_ref, out_ref,
          tbl_v, x0_v, x1_v, o0_v, o1_v,
          sx0, sx1, sw0, sw1, sr0, sr1):
    cid = lax.axis_index("c")
    sid = lax.axis_index("s")
    base = (sid * NCORE + cid) * PTS

    x_bufs = (x0_v, x1_v)
    o_bufs = (o0_v, o1_v)
    sx = (sx0, sx1)
    sw = (sw0, sw1)
    sr = (sr0, sr1)

    def x_src(k):
        return xt_ref.at[:, pl.ds(base + k * C, C)]

    def out_cols(k):
        return out_ref.at[:, pl.ds(base + k * C, C)]

    def compute(x_v, o_v, levels, off0):
        def vec_body(i, _):
            p = i * 16
            xs = x_v[0, pl.ds(p, 16)]
            ys = x_v[1, pl.ds(p, 16)]
            for l in levels:
                w = WPAD[l]
                c_l = SL[l] / 2.0
                k_l = (_offs[l] - off0) - 255 * w - 255
                ix = xs * c_l + 255.5
                iy = ys * c_l + 255.5
                x0 = ix.astype(jnp.int32)
                y0 = iy.astype(jnp.int32)
                fx = ix - x0.astype(jnp.float32)
                fy = iy - y0.astype(jnp.float32)
                gx = 1.0 - fx
                gy = 1.0 - fy
                i00 = y0 * w + x0 + k_l
                v00 = plsc.load_gather(tbl_v, [i00])
                v01 = plsc.load_gather(tbl_v, [i00 + 1])
                v10 = plsc.load_gather(tbl_v, [i00 + w])
                v11 = plsc.load_gather(tbl_v, [i00 + (w + 1)])
                w00 = gx * gy
                w01 = fx * gy
                w10 = gx * fy
                w11 = fx * fy
                a0 = ((w00 * _f32_lo(v00) + w01 * _f32_lo(v01))
                      + (w10 * _f32_lo(v10) + w11 * _f32_lo(v11)))
                a1 = ((w00 * _f32_hi(v00) + w01 * _f32_hi(v01))
                      + (w10 * _f32_hi(v10) + w11 * _f32_hi(v11)))
                o_v[l, pl.ds(p, 16)] = a0
                o_v[L + l, pl.ds(p, 16)] = a1
            return 0

        lax.fori_loop(0, C // 16, vec_body, 0)

    def run_pass(tbl_hbm, nwords, levels, off0, rmw):
        pltpu.sync_copy(tbl_hbm, tbl_v.at[pl.ds(0, nwords)])
        pltpu.async_copy(x_src(0), x_bufs[0], sx[0])
        pltpu.async_copy(x_src(1), x_bufs[1], sx[1])
        if rmw:
            pltpu.async_copy(out_cols(0), o_bufs[0], sr[0])
            pltpu.async_copy(out_cols(1), o_bufs[1], sr[1])

        def half(k, b):
            pltpu.make_async_copy(x_src(k), x_bufs[b], sx[b]).wait()
            if rmw:
                pltpu.make_async_copy(out_cols(k), o_bufs[b], sr[b]).wait()
            else:
                @pl.when(k >= 2)
                def _():
                    pltpu.make_async_copy(o_bufs[b], out_cols(k - 2),
                                          sw[b]).wait()
            compute(x_bufs[b], o_bufs[b], levels, off0)
            pltpu.async_copy(o_bufs[b], out_cols(k), sw[b])

            @pl.when(k + 2 <= NCHUNK - 1)
            def _():
                pltpu.async_copy(x_src(k + 2), x_bufs[b], sx[b])
                if rmw:
                    pltpu.make_async_copy(o_bufs[b], out_cols(k), sw[b]).wait()
                    pltpu.async_copy(out_cols(k + 2), o_bufs[b], sr[b])

        def jbody(j, _):
            half(j * 2, 0)
            half(j * 2 + 1, 1)
            return 0

        lax.fori_loop(0, NJ, jbody, 0)
        pltpu.make_async_copy(o_bufs[0], out_cols(NCHUNK - 2), sw[0]).wait()
        pltpu.make_async_copy(o_bufs[1], out_cols(NCHUNK - 1), sw[1]).wait()

    def zero_body(i, _):
        p = i * 16
        z = jnp.zeros((16,), jnp.float32)
        for cc in list(range(14)) + list(range(16, 30)):
            o0_v[cc, pl.ds(p, 16)] = z
            o1_v[cc, pl.ds(p, 16)] = z
        return 0

    lax.fori_loop(0, C // 16, zero_body, 0)
    run_pass(tblb_ref, NB, PASS_B, _offs[14], rmw=False)
    run_pass(tbla_ref, NA, PASS_A, 0, rmw=True)


@jax.jit
def kernel(x, grid_table):
    tbl_a, tbl_b = _pack_tables(grid_table)
    xt = x.T
    mesh = plsc.VectorSubcoreMesh(core_axis_name="c", subcore_axis_name="s")
    fn = pl.kernel(
        _body,
        out_type=jax.ShapeDtypeStruct((NCOL, B), jnp.float32),
        mesh=mesh,
        compiler_params=pltpu.CompilerParams(needs_layout_passes=False),
        scratch_types=[
            pltpu.VMEM((NB,), jnp.int32),
            pltpu.VMEM((2, C), jnp.float32),
            pltpu.VMEM((2, C), jnp.float32),
            pltpu.VMEM((NCOL, C), jnp.float32),
            pltpu.VMEM((NCOL, C), jnp.float32),
            pltpu.SemaphoreType.DMA,
            pltpu.SemaphoreType.DMA,
            pltpu.SemaphoreType.DMA,
            pltpu.SemaphoreType.DMA,
            pltpu.SemaphoreType.DMA,
            pltpu.SemaphoreType.DMA,
        ],
    )
    return fn(xt, tbl_a, tbl_b).T
